# Initial kernel scaffold; baseline (speedup 1.0000x reference)
import functools

import jax
import jax.numpy as jnp
from jax import lax
from jax.experimental import pallas as pl
from jax.experimental.pallas import tpu as pltpu
from jax.experimental.pallas import tpu_sc as plsc

NUM_EMBEDDINGS = 1000000
EMBED_DIM = 32
BATCH = 16384
HIST_LEN = 50

_B = BATCH * HIST_LEN
_NC = 2
_NS = 16
_NW = _NC * _NS
_PER_W = _B // _NW
_CHUNK = 1024
_STEPS = _PER_W // _CHUNK


def _make_sc_gather():
    mesh = plsc.VectorSubcoreMesh(core_axis_name="c", subcore_axis_name="s")

    @functools.partial(
        pl.kernel,
        mesh=mesh,
        out_type=jax.ShapeDtypeStruct((_B, EMBED_DIM), jnp.float32),
        scratch_types=[
            pltpu.VMEM((_CHUNK,), jnp.int32),
            pltpu.VMEM((_CHUNK, EMBED_DIM), jnp.float32),
            pltpu.SemaphoreType.DMA,
        ],
    )
    def gather_kernel(idx_hbm, table_hbm, out_hbm, idx_v, rows_v, sem):
        wid = lax.axis_index("s") * _NC + lax.axis_index("c")
        base = wid * _PER_W

        def step(i, carry):
            off = base + i * _CHUNK
            pltpu.sync_copy(idx_hbm.at[pl.ds(off, _CHUNK)], idx_v)
            pltpu.async_copy(table_hbm.at[idx_v], rows_v, sem).wait()
            pltpu.sync_copy(rows_v, out_hbm.at[pl.ds(off, _CHUNK)])
            return carry

        lax.fori_loop(0, _STEPS, step, 0)

        return gather_kernel

    return gather_kernel


_gather = _make_sc_gather()


@jax.jit
def kernel(x, emb):
    idx = x.reshape(_B).astype(jnp.int32)
    out = _gather(idx, emb)
    return out.reshape(BATCH, HIST_LEN, EMBED_DIM)


# speedup vs baseline: 1.0939x; 1.0939x over previous
import functools

import jax
import jax.numpy as jnp
from jax import lax
from jax.experimental import pallas as pl
from jax.experimental.pallas import tpu as pltpu
from jax.experimental.pallas import tpu_sc as plsc

NUM_EMBEDDINGS = 1000000
EMBED_DIM = 32
BATCH = 16384
HIST_LEN = 50

_B = BATCH * HIST_LEN
_NC = 2
_NS = 16
_NW = _NC * _NS
_PER_W = _B // _NW
_CHUNK = 1024
_STEPS = _PER_W // _CHUNK


def _make_sc_gather():
    mesh = plsc.VectorSubcoreMesh(core_axis_name="c", subcore_axis_name="s")

    @functools.partial(
        pl.kernel,
        mesh=mesh,
        out_type=jax.ShapeDtypeStruct((_B, EMBED_DIM), jnp.float32),
        scratch_types=[
            pltpu.VMEM((_CHUNK,), jnp.int32),
            pltpu.VMEM((_CHUNK, EMBED_DIM), jnp.float32),
            pltpu.SemaphoreType.DMA,
        ],
        compiler_params=pltpu.CompilerParams(use_tc_tiling_on_sc=False),
    )
    def gather_kernel(idx_hbm, table_hbm, out_hbm, idx_v, rows_v, sem):
        wid = lax.axis_index("s") * _NC + lax.axis_index("c")
        base = wid * _PER_W

        def step(i, carry):
            off = base + i * _CHUNK
            pltpu.sync_copy(idx_hbm.at[pl.ds(off, _CHUNK)], idx_v)
            pltpu.async_copy(table_hbm.at[idx_v], rows_v, sem).wait()
            pltpu.sync_copy(rows_v, out_hbm.at[pl.ds(off, _CHUNK)])
            return carry

        lax.fori_loop(0, _STEPS, step, 0)

    return gather_kernel


_gather = _make_sc_gather()


@jax.jit
def kernel(x, emb):
    idx = x.reshape(_B).astype(jnp.int32)
    out = _gather(idx, emb)
    return out.reshape(BATCH, HIST_LEN, EMBED_DIM)


# speedup vs baseline: 1.1112x; 1.0158x over previous
import functools

import jax
import jax.numpy as jnp
from jax import lax
from jax.experimental import pallas as pl
from jax.experimental.pallas import tpu as pltpu
from jax.experimental.pallas import tpu_sc as plsc

NUM_EMBEDDINGS = 1000000
EMBED_DIM = 32
BATCH = 16384
HIST_LEN = 50

_B = BATCH * HIST_LEN
_NC = 2
_NS = 16
_NW = _NC * _NS
_PER_W = _B // _NW
_K = 4
_CHUNK = 640
_STEPS = _PER_W // (_K * _CHUNK)


def _make_sc_gather():
    mesh = plsc.VectorSubcoreMesh(core_axis_name="c", subcore_axis_name="s")

    @functools.partial(
        pl.kernel,
        mesh=mesh,
        out_type=jax.ShapeDtypeStruct((_B, EMBED_DIM), jnp.float32),
        scratch_types=[
            pltpu.VMEM((_PER_W,), jnp.int32),
            pltpu.VMEM((_K, _CHUNK, EMBED_DIM), jnp.float32),
            pltpu.SemaphoreType.DMA,
            pltpu.SemaphoreType.DMA,
            pltpu.SemaphoreType.DMA,
            pltpu.SemaphoreType.DMA,
        ],
        compiler_params=pltpu.CompilerParams(use_tc_tiling_on_sc=False),
    )
    def gather_kernel(idx_hbm, table_hbm, out_hbm, idx_v, rows_v, s0, s1, s2, s3):
        wid = lax.axis_index("s") * _NC + lax.axis_index("c")
        base = wid * _PER_W
        sems = [s0, s1, s2, s3]

        pltpu.sync_copy(idx_hbm.at[pl.ds(base, _PER_W)], idx_v)

        def step(i, carry):
            offs = i * (_K * _CHUNK)
            handles = [
                pltpu.async_copy(
                    table_hbm.at[idx_v.at[pl.ds(offs + k * _CHUNK, _CHUNK)]],
                    rows_v.at[k],
                    sems[k],
                )
                for k in range(_K)
            ]
            for k in range(_K):
                handles[k].wait()
                pltpu.sync_copy(
                    rows_v.at[k],
                    out_hbm.at[pl.ds(base + offs + k * _CHUNK, _CHUNK)],
                )
            return carry

        lax.fori_loop(0, _STEPS, step, 0)

    return gather_kernel


_gather = _make_sc_gather()


@jax.jit
def kernel(x, emb):
    idx = x.reshape(_B).astype(jnp.int32)
    out = _gather(idx, emb)
    return out.reshape(BATCH, HIST_LEN, EMBED_DIM)


# speedup vs baseline: 1.2790x; 1.1510x over previous
import functools

import jax
import jax.numpy as jnp
from jax import lax
from jax.experimental import pallas as pl
from jax.experimental.pallas import tpu as pltpu
from jax.experimental.pallas import tpu_sc as plsc

NUM_EMBEDDINGS = 1000000
EMBED_DIM = 32
BATCH = 16384
HIST_LEN = 50

_B = BATCH * HIST_LEN
_NC = 2
_NS = 16
_NW = _NC * _NS
_PER_W = _B // _NW
_K = 4
_CHUNK = 640
_STEPS = _PER_W // (_K * _CHUNK)


def _make_sc_gather():
    mesh = plsc.VectorSubcoreMesh(core_axis_name="c", subcore_axis_name="s")

    @functools.partial(
        pl.kernel,
        mesh=mesh,
        out_type=jax.ShapeDtypeStruct((_B, EMBED_DIM), jnp.float32),
        scratch_types=[
            pltpu.VMEM((_PER_W,), jnp.int32),
            pltpu.VMEM((_K, _CHUNK, EMBED_DIM), jnp.float32),
            pltpu.SemaphoreType.DMA,
            pltpu.SemaphoreType.DMA,
            pltpu.SemaphoreType.DMA,
            pltpu.SemaphoreType.DMA,
        ],
        compiler_params=pltpu.CompilerParams(use_tc_tiling_on_sc=False),
    )
    def gather_kernel(idx_hbm, table_hbm, out_hbm, idx_v, rows_v, s0, s1, s2, s3):
        wid = lax.axis_index("s") * _NC + lax.axis_index("c")
        base = wid * _PER_W
        sems = [s0, s1, s2, s3]

        pltpu.sync_copy(idx_hbm.at[pl.ds(base, _PER_W)], idx_v)

        def xf(j, c):
            v = idx_v[pl.ds(j * 16, 16)]
            idx_v[pl.ds(j * 16, 16)] = (
                lax.shift_left(lax.shift_right_logical(v, 13), 13)
                + lax.shift_left(jnp.bitwise_and(v, 2047), 2)
                + jnp.bitwise_and(lax.shift_right_logical(v, 11), 3))
            return c

        lax.fori_loop(0, _PER_W // 16, xf, 0)

        def step(i, carry):
            offs = i * (_K * _CHUNK)
            handles = [
                pltpu.async_copy(
                    table_hbm.at[idx_v.at[pl.ds(offs + k * _CHUNK, _CHUNK)]],
                    rows_v.at[k],
                    sems[k],
                )
                for k in range(_K)
            ]
            for k in range(_K):
                handles[k].wait()
                pltpu.sync_copy(
                    rows_v.at[k],
                    out_hbm.at[pl.ds(base + offs + k * _CHUNK, _CHUNK)],
                )
            return carry

        lax.fori_loop(0, _STEPS, step, 0)

    return gather_kernel


_gather = _make_sc_gather()

_K1_COLS = 8192
_K1_SUB = _K1_COLS // 4
_K1_GRID = -(-NUM_EMBEDDINGS // _K1_COLS)
_R_ROWS = _K1_GRID * _K1_SUB


def _repack_body(x_ref, o_ref):
    x = x_ref[...]
    for a in range(4):
        o_ref[:, 32 * a:32 * (a + 1)] = x[:, _K1_SUB * a:_K1_SUB * (a + 1)].T


_repack = pl.pallas_call(
    _repack_body,
    grid=(_K1_GRID,),
    in_specs=[pl.BlockSpec((EMBED_DIM, _K1_COLS), lambda j: (0, j))],
    out_specs=pl.BlockSpec((_K1_SUB, 128), lambda j: (j, 0)),
    out_shape=jax.ShapeDtypeStruct((_R_ROWS, 128), jnp.float32),
)


@jax.jit
def kernel(x, emb):
    idx = x.reshape(_B).astype(jnp.int32)
    packed = _repack(emb.T)
    emb_lin = packed.reshape(4 * _R_ROWS, EMBED_DIM)
    out = _gather(idx, emb_lin)
    return out.reshape(BATCH, HIST_LEN, EMBED_DIM)


# speedup vs baseline: 2.7353x; 2.1387x over previous
import functools

import jax
import jax.numpy as jnp
from jax import lax
from jax.experimental import pallas as pl
from jax.experimental.pallas import tpu as pltpu
from jax.experimental.pallas import tpu_sc as plsc

NUM_EMBEDDINGS = 1000000
EMBED_DIM = 32
BATCH = 16384
HIST_LEN = 50

_B = BATCH * HIST_LEN
_NC = 2
_NS = 16
_NW = _NC * _NS
_PER_W = _B // _NW
_K = 4
_CHUNK = 640
_STEPS = _PER_W // (_K * _CHUNK)


def _make_sc_gather():
    mesh = plsc.VectorSubcoreMesh(core_axis_name="c", subcore_axis_name="s")

    @functools.partial(
        pl.kernel,
        mesh=mesh,
        out_type=jax.ShapeDtypeStruct((_B, EMBED_DIM), jnp.float32),
        scratch_types=[
            pltpu.VMEM((_PER_W,), jnp.int32),
            pltpu.VMEM((_K, _CHUNK, EMBED_DIM), jnp.float32),
            pltpu.SemaphoreType.DMA,
            pltpu.SemaphoreType.DMA,
            pltpu.SemaphoreType.DMA,
            pltpu.SemaphoreType.DMA,
        ],
        compiler_params=pltpu.CompilerParams(use_tc_tiling_on_sc=False),
    )
    def gather_kernel(idx_hbm, table_hbm, out_hbm, idx_v, rows_v, s0, s1, s2, s3):
        wid = lax.axis_index("s") * _NC + lax.axis_index("c")
        base = wid * _PER_W
        sems = [s0, s1, s2, s3]

        pltpu.sync_copy(idx_hbm.at[pl.ds(base, _PER_W)], idx_v)

        def xf(j, c):
            v = idx_v[pl.ds(j * 16, 16)]
            idx_v[pl.ds(j * 16, 16)] = (
                lax.shift_left(lax.shift_right_logical(v, 13), 13)
                + lax.shift_left(jnp.bitwise_and(v, 2047), 2)
                + jnp.bitwise_and(lax.shift_right_logical(v, 11), 3))
            return c

        lax.fori_loop(0, _PER_W // 16, xf, 0)

        def step(i, carry):
            offs = i * (_K * _CHUNK)
            handles = [
                pltpu.async_copy(
                    table_hbm.at[idx_v.at[pl.ds(offs + k * _CHUNK, _CHUNK)]],
                    rows_v.at[k],
                    sems[k],
                )
                for k in range(_K)
            ]
            for k in range(_K):
                handles[k].wait()
                pltpu.sync_copy(
                    rows_v.at[k],
                    out_hbm.at[pl.ds(base + offs + k * _CHUNK, _CHUNK)],
                )
            return carry

        lax.fori_loop(0, _STEPS, step, 0)

    return gather_kernel


_gather = _make_sc_gather()

_K1_COLS = 8192
_K1_SUB = _K1_COLS // 4
_K1_GRID = -(-NUM_EMBEDDINGS // _K1_COLS)
_R_ROWS = _K1_GRID * _K1_SUB


def _repack_body(x_ref, o_ref):
    x = x_ref[...]
    for a in range(4):
        o_ref[:, 32 * a:32 * (a + 1)] = x[:, _K1_SUB * a:_K1_SUB * (a + 1)].T


_repack = pl.pallas_call(
    _repack_body,
    grid=(_K1_GRID,),
    in_specs=[pl.BlockSpec((EMBED_DIM, _K1_COLS), lambda j: (0, j))],
    out_specs=pl.BlockSpec((_K1_SUB, 128), lambda j: (j, 0)),
    out_shape=jax.ShapeDtypeStruct((_R_ROWS, 128), jnp.float32),
)


_BQ = BATCH // 4


def _finish_body(x_ref, o_ref):
    x = x_ref[...]
    for a in range(4):
        o_ref[0, :, _BQ * a:_BQ * (a + 1)] = x[:, 32 * a:32 * (a + 1)].T


_finish = pl.pallas_call(
    _finish_body,
    grid=(HIST_LEN,),
    in_specs=[pl.BlockSpec((_BQ, 128), lambda h: (h, 0))],
    out_specs=pl.BlockSpec((1, EMBED_DIM, BATCH), lambda h: (h, 0, 0)),
    out_shape=jax.ShapeDtypeStruct((HIST_LEN, EMBED_DIM, BATCH), jnp.float32),
)


@jax.jit
def kernel(x, emb):
    idx = (x.T.reshape(HIST_LEN, 4, _BQ).transpose(0, 2, 1)
           .reshape(_B).astype(jnp.int32))
    packed = _repack(emb.T)
    emb_lin = packed.reshape(4 * _R_ROWS, EMBED_DIM)
    out = _gather(idx, emb_lin)
    t = _finish(out.reshape(HIST_LEN * _BQ, 4 * EMBED_DIM))
    return jnp.transpose(t, (2, 0, 1))
